# baseline (speedup 1.0000x reference)
import functools

import jax
import jax.numpy as jnp
from jax import lax
from jax.experimental import pallas as pl
from jax.experimental.pallas import tpu as pltpu
from jax.experimental.pallas import tpu_sc as plsc

L = 2048
H = 16
NSHIFT = 128
TW = 4096
TW_S = 3968
NC = 2
NS = 16
HEADS_PER_PHASE = 4
N_PHASES = 2
GROUPS = L // NSHIFT


def _sc_body(p_hbm, out_hbm, sp, sem_load, sem_w):
    cid = lax.axis_index("c")
    sid = lax.axis_index("s")
    slab = lax.rem(sid, HEADS_PER_PHASE)
    quarter = sid // HEADS_PER_PHASE

    for phase in range(N_PHASES):
        h = cid * (H // NC) + phase * HEADS_PER_PHASE + slab

        rb = pl.multiple_of(quarter * (NSHIFT // 4), 8)
        pltpu.async_copy(
            p_hbm.at[h, pl.ds(rb, NSHIFT // 4), pl.ds(0, TW_S)],
            sp.at[slab, pl.ds(rb, NSHIFT // 4), :],
            sem_load,
        ).wait()
        plsc.subcore_barrier()

        for k in range(GROUPS // 4):
            g = quarter + 4 * k
            i0 = pl.multiple_of(g * NSHIFT, NSHIFT)
            a = pl.multiple_of((L - NSHIFT) - i0, NSHIFT)
            pltpu.async_copy(
                sp.at[slab, :, pl.ds(a, L)],
                out_hbm.at[0, h, pl.ds(i0, NSHIFT)],
                sem_w,
            )
        for _ in range(GROUPS // 4):
            pltpu.make_async_copy(
                sp.at[0, :, pl.ds(0, L)],
                out_hbm.at[0, 0, pl.ds(0, NSHIFT)],
                sem_w,
            ).wait()
        plsc.subcore_barrier()


@jax.jit
def _run_sc(p):
    mesh = plsc.VectorSubcoreMesh(
        core_axis_name="c", subcore_axis_name="s", num_cores=NC, num_subcores=NS
    )
    return pl.kernel(
        _sc_body,
        out_type=jax.ShapeDtypeStruct((1, H, L, L), jnp.float32),
        mesh=mesh,
        scratch_types=[
            pltpu.VMEM_SHARED((HEADS_PER_PHASE, NSHIFT, TW_S), jnp.float32),
            pltpu.SemaphoreType.DMA,
            pltpu.SemaphoreType.DMA,
        ],
    )(p)


def kernel(x, bias_table):
    del x
    tt = jnp.transpose(jnp.pad(bias_table, ((0, NSHIFT + TW - (2 * L - 1)), (0, 0))))
    p = jnp.stack(
        [lax.slice(tt, (0, NSHIFT - 1 - q), (H, NSHIFT - 1 - q + TW))
         for q in range(NSHIFT)], axis=1)
    return _run_sc(p)
